# baseline (speedup 1.0000x reference)
import dataclasses
import functools

import jax
import jax.numpy as jnp
from jax import lax
from jax.experimental import pallas as pl
from jax.experimental.pallas import tpu as pltpu
from jax.experimental.pallas import tpu_sc as plsc

N = 10000
NPAD = 10240
E = 320000
D = 128
EPS = 1e-5
BNS = 1.0 / (1.0 + EPS) ** 0.5

NC, NS, L = 2, 16, 16
NW = NC * NS
EPW = E // NW
ST = 128
SUP = ST
SUPT = 78
NSUP = SUPT
NEXTRA = (E - NW * SUPT * SUP) // ST
HR = NPAD // 128
ICH = 2000
NICH = EPW // ICH

_MESH = plsc.VectorSubcoreMesh(core_axis_name="c", subcore_axis_name="s")
_CP = pltpu.CompilerParams()
if "needs_layout_passes" in pltpu.CompilerParams.__dataclass_fields__:
    _CP = dataclasses.replace(_CP, needs_layout_passes=False)

RPT_A = 624
RPT_B = N - 15 * RPT_A
TB = 400
NTB = N // TB


def _prep_body(ei_hbm, outd_hbm, outg_hbm, hist, dinv_v, dbuf, idxrows,
               ib_s, ib_d, sdeg, sg, isem):
    c = lax.axis_index("c")
    s = lax.axis_index("s")

    zf = jnp.zeros((L,), jnp.float32)
    ones = zf + 1.0
    lane_iota = lax.iota(jnp.int32, L)

    def zero_hist():
        @pl.loop(0, HR)
        def _(i):
            @pl.loop(0, 128 // L)
            def _(k):
                hist[i, pl.ds(k * L, L)] = zf

    zero_hist()

    @pl.loop(0, HR // L)
    def _(k):
        idxrows[0, pl.ds(k * L, L)] = k * L + lane_iota

    @pl.when(s == 0)
    def _():
        pltpu.sync_copy(hist, sdeg)
        pltpu.sync_copy(hist, sg)
    plsc.subcore_barrier()

    DCH = E // NS // ICH

    def start_dst(ci, p):
        pltpu.async_copy(ei_hbm.at[pl.ds(E + s * (E // NS) + ci * ICH, ICH)],
                         ib_d.at[pl.ds(p * ICH, ICH)], isem.at[p])

    start_dst(0, 0)

    @pl.loop(0, DCH)
    def _(ci):
        p = lax.rem(ci, 2)
        pltpu.make_async_copy(ei_hbm.at[pl.ds(0, ICH)],
                              ib_d.at[pl.ds(p * ICH, ICH)],
                              isem.at[p]).wait()

        @pl.when(ci + 1 < DCH)
        def _():
            start_dst(ci + 1, 1 - p)

        @pl.loop(0, ICH // L)
        def _(i):
            key = ib_d[pl.ds(p * ICH + i * L, L)]
            plsc.addupdate_scatter(
                hist,
                [lax.shift_right_logical(key, 7), lax.bitwise_and(key, 127)],
                ones,
            )

    pltpu.sync_copy(hist, sdeg.at[idxrows.at[0]], add=True)
    plsc.subcore_barrier()

    pltpu.sync_copy(sdeg.at[pl.ds(5 * s, 5)], dbuf)

    @pl.loop(0, 5)
    def _(i):
        @pl.loop(0, 128 // L)
        def _(k):
            d = dbuf[i, pl.ds(k * L, L)] + 1.0
            bits = plsc.bitcast(d, jnp.int32)
            y = plsc.bitcast(jnp.full((L,), 0x5F3759DF, jnp.int32)
                             - lax.shift_right_logical(bits, 1), jnp.float32)
            hd = d * 0.5
            y = y * (1.5 - hd * y * y)
            y = y * (1.5 - hd * y * y)
            y = y * (1.5 - hd * y * y)
            y = y * (1.5 - hd * y * y)
            dbuf[i, pl.ds(k * L, L)] = y

    pltpu.sync_copy(dbuf, sdeg.at[pl.ds(5 * s, 5)])
    plsc.subcore_barrier()

    pltpu.sync_copy(sdeg, dinv_v)

    @pl.when((s == 0) & (c == 0))
    def _():
        pltpu.sync_copy(sdeg, outd_hbm)

    zero_hist()

    GCH = E // NC // NS // ICH
    gbase = c * (E // NC) + s * (E // NC // NS)

    def start_pair(ci, p):
        pltpu.async_copy(ei_hbm.at[pl.ds(gbase + ci * ICH, ICH)],
                         ib_s.at[pl.ds(p * ICH, ICH)], isem.at[p])
        pltpu.async_copy(ei_hbm.at[pl.ds(E + gbase + ci * ICH, ICH)],
                         ib_d.at[pl.ds(p * ICH, ICH)], isem.at[p])

    start_pair(0, 0)

    @pl.loop(0, GCH)
    def _(ci):
        p = lax.rem(ci, 2)
        pltpu.make_async_copy(ei_hbm.at[pl.ds(0, ICH)],
                              ib_s.at[pl.ds(p * ICH, ICH)],
                              isem.at[p]).wait()
        pltpu.make_async_copy(ei_hbm.at[pl.ds(0, ICH)],
                              ib_d.at[pl.ds(p * ICH, ICH)],
                              isem.at[p]).wait()

        @pl.when(ci + 1 < GCH)
        def _():
            start_pair(ci + 1, 1 - p)

        @pl.loop(0, ICH // L)
        def _(i):
            ks = ib_s[pl.ds(p * ICH + i * L, L)]
            kd = ib_d[pl.ds(p * ICH + i * L, L)]
            vals = plsc.load_gather(
                dinv_v,
                [lax.shift_right_logical(kd, 7), lax.bitwise_and(kd, 127)])
            plsc.addupdate_scatter(
                hist,
                [lax.shift_right_logical(ks, 7), lax.bitwise_and(ks, 127)],
                vals,
            )

    pltpu.sync_copy(hist, sg.at[idxrows.at[0]], add=True)
    plsc.subcore_barrier()

    @pl.when(s == 0)
    def _():
        pltpu.sync_copy(sg, outg_hbm.at[c])


def _prep_sc(ei):
    return pl.kernel(
        _prep_body,
        out_type=(jax.ShapeDtypeStruct((HR, 128), jnp.float32),
                  jax.ShapeDtypeStruct((NC, HR, 128), jnp.float32)),
        mesh=_MESH,
        compiler_params=_CP,
        scratch_types=[
            pltpu.VMEM((HR, 128), jnp.float32),
            pltpu.VMEM((HR, 128), jnp.float32),
            pltpu.VMEM((5, 128), jnp.float32),
            pltpu.VMEM((1, HR), jnp.int32),
            pltpu.VMEM((2 * ICH,), jnp.int32),
            pltpu.VMEM((2 * ICH,), jnp.int32),
            pltpu.VMEM_SHARED((HR, 128), jnp.float32),
            pltpu.VMEM_SHARED((HR, 128), jnp.float32),
            pltpu.SemaphoreType.DMA((2,)),
        ],
    )(ei)


def _conv_body(t_hbm, ei_hbm, out_hbm, sidx, didx, rows, acc, isem, gsem, ssem):
    c = lax.axis_index("c")
    s = lax.axis_index("s")
    w = c * NS + s
    base = w * SUPT * SUP

    @pl.when(s < NS - 1)
    def _():
        pltpu.sync_copy(t_hbm.at[pl.ds(s * RPT_A, RPT_A)],
                        acc.at[pl.ds(s * RPT_A, RPT_A)])

    @pl.when(s == NS - 1)
    def _():
        pltpu.sync_copy(t_hbm.at[pl.ds(15 * RPT_A, RPT_B)],
                        acc.at[pl.ds(15 * RPT_A, RPT_B)])
    plsc.subcore_barrier()

    def start_idx(j, m):
        pltpu.async_copy(ei_hbm.at[pl.ds(base + j * SUP, SUP)],
                         sidx.at[m], isem.at[m])
        pltpu.async_copy(ei_hbm.at[pl.ds(E + base + j * SUP, ST)],
                         didx.at[m, 0], isem.at[m])

    def wait_idx(m):
        pltpu.make_async_copy(ei_hbm.at[pl.ds(0, SUP)], sidx.at[m],
                              isem.at[m]).wait()
        pltpu.make_async_copy(ei_hbm.at[pl.ds(0, ST)], didx.at[m, 0],
                              isem.at[m]).wait()

    def start_gather(m, r):
        pltpu.async_copy(t_hbm.at[sidx.at[m]], rows.at[r], gsem.at[r])

    def wait_gather(r):
        pltpu.make_async_copy(t_hbm.at[pl.ds(0, ST)], rows.at[r],
                              gsem.at[r]).wait()

    def start_scatter(m, r):
        pltpu.async_copy(rows.at[r], acc.at[didx.at[m, 0]], ssem.at[r],
                         add=True)

    def wait_scatter(m, r):
        pltpu.make_async_copy(rows.at[r], acc.at[didx.at[m, 0]],
                              ssem.at[r]).wait()

    start_idx(0, 0)
    start_idx(1, 1)

    @pl.loop(0, NSUP)
    def _(j):
        m = lax.rem(j, 4)
        r = lax.rem(j, 3)
        wait_idx(m)

        @pl.when(j >= 3)
        def _():
            wait_scatter(lax.rem(j + 1, 4), r)
        start_gather(m, r)

        @pl.when(j >= 2)
        def _():
            r2 = lax.rem(j + 1, 3)
            m2 = lax.rem(j + 2, 4)
            wait_gather(r2)
            start_scatter(m2, r2)

        @pl.when(j + 2 < NSUP)
        def _():
            start_idx(j + 2, lax.rem(j + 2, 4))

    wait_gather((NSUP - 2) % 3)
    start_scatter((NSUP - 2) % 4, (NSUP - 2) % 3)
    wait_gather((NSUP - 1) % 3)
    start_scatter((NSUP - 1) % 4, (NSUP - 1) % 3)
    wait_scatter((NSUP - 3) % 4, (NSUP - 3) % 3)
    wait_scatter((NSUP - 2) % 4, (NSUP - 2) % 3)
    wait_scatter((NSUP - 1) % 4, (NSUP - 1) % 3)

    @pl.when(w < NEXTRA)
    def _():
        xbase = NW * SUPT * SUP + w * ST
        pltpu.sync_copy(ei_hbm.at[pl.ds(xbase, ST)], sidx.at[0])
        pltpu.sync_copy(ei_hbm.at[pl.ds(E + xbase, ST)], didx.at[0, 0])
        pltpu.sync_copy(t_hbm.at[sidx.at[0]], rows.at[0])
        pltpu.sync_copy(rows.at[0], acc.at[didx.at[0, 0]], add=True)

    plsc.subcore_barrier()

    @pl.when(s < NS - 1)
    def _():
        pltpu.sync_copy(acc.at[pl.ds(s * RPT_A, RPT_A)],
                        out_hbm.at[c, pl.ds(s * RPT_A, RPT_A)])

    @pl.when(s == NS - 1)
    def _():
        pltpu.sync_copy(acc.at[pl.ds(15 * RPT_A, RPT_B)],
                        out_hbm.at[c, pl.ds(15 * RPT_A, RPT_B)])


def _conv_sc(t, ei):
    return pl.kernel(
        _conv_body,
        out_type=jax.ShapeDtypeStruct((NC, N, D), jnp.float32),
        mesh=_MESH,
        compiler_params=_CP,
        scratch_types=[
            pltpu.VMEM((4, SUP), jnp.int32),
            pltpu.VMEM((4, 1, ST), jnp.int32),
            pltpu.VMEM((3, SUP, D), jnp.float32),
            pltpu.VMEM_SHARED((N, D), jnp.float32),
            pltpu.SemaphoreType.DMA((4,)),
            pltpu.SemaphoreType.DMA((3,)),
            pltpu.SemaphoreType.DMA((3,)),
        ],
    )(t, ei)


def _tc1_body(dinv_ref, x_ref, w1_ref, t1_ref):
    t1_ref[...] = dinv_ref[...] * jnp.dot(x_ref[...], w1_ref[...],
                                          preferred_element_type=jnp.float32)


def _tc1(dinv, x, W1):
    return pl.pallas_call(
        _tc1_body,
        grid=(NTB,),
        in_specs=[
            pl.BlockSpec((TB, 1), lambda i: (i, 0)),
            pl.BlockSpec((TB, D), lambda i: (i, 0)),
            pl.BlockSpec((D, D), lambda i: (0, 0)),
        ],
        out_specs=pl.BlockSpec((TB, D), lambda i: (i, 0)),
        out_shape=jax.ShapeDtypeStruct((N, D), jnp.float32),
    )(dinv, x, W1)


def _tc2_body(acc_ref, t1_ref, dinv_ref, w2_ref, b1_ref, g1_ref, be1_ref,
              t2_ref):
    dinv = dinv_ref[...]
    u = acc_ref[0] + acc_ref[1] - t1_ref[...]
    pre = dinv * u + b1_ref[...]
    h1 = jnp.maximum(pre * (g1_ref[...] * BNS) + be1_ref[...], 0.0)
    t2_ref[...] = dinv * jnp.dot(h1, w2_ref[...],
                                 preferred_element_type=jnp.float32)


def _tc2(acc, t1, dinv, W2, b1, g1, be1):
    return pl.pallas_call(
        _tc2_body,
        grid=(NTB,),
        in_specs=[
            pl.BlockSpec((NC, TB, D), lambda i: (0, i, 0)),
            pl.BlockSpec((TB, D), lambda i: (i, 0)),
            pl.BlockSpec((TB, 1), lambda i: (i, 0)),
            pl.BlockSpec((D, D), lambda i: (0, 0)),
            pl.BlockSpec((1, D), lambda i: (0, 0)),
            pl.BlockSpec((1, D), lambda i: (0, 0)),
            pl.BlockSpec((1, D), lambda i: (0, 0)),
        ],
        out_specs=pl.BlockSpec((TB, D), lambda i: (i, 0)),
        out_shape=jax.ShapeDtypeStruct((N, D), jnp.float32),
    )(acc, t1, dinv, W2, b1, g1, be1)


def _tc3_body(acc_ref, t2_ref, dinv_ref, gp_ref, b2_ref, g2_ref, be2_ref,
              w3_ref, b3_ref, g3_ref, be3_ref, out_ref, ysum):
    i = pl.program_id(0)
    dinv = dinv_ref[...]
    u = acc_ref[0] + acc_ref[1] - t2_ref[...]
    pre = dinv * u + b2_ref[...]
    h2 = jnp.maximum(pre * (g2_ref[...] * BNS) + be2_ref[...], 0.0)
    g = gp_ref[0] + gp_ref[1]
    w = dinv * (g + dinv) * (1.0 / N)
    part = jnp.sum(w * h2, axis=0, keepdims=True)

    @pl.when(i == 0)
    def _():
        ysum[...] = jnp.zeros_like(ysum)

    ysum[...] += part

    @pl.when(i == NTB - 1)
    def _():
        y = jnp.dot(ysum[...], w3_ref[...], preferred_element_type=jnp.float32)
        out_ref[...] = (y + b3_ref[...]) * (g3_ref[...] * BNS) + be3_ref[...]


def _tc3(acc, t2, dinv, gp, b2, g2, be2, W3, b3, g3, be3):
    return pl.pallas_call(
        _tc3_body,
        grid=(NTB,),
        in_specs=[
            pl.BlockSpec((NC, TB, D), lambda i: (0, i, 0)),
            pl.BlockSpec((TB, D), lambda i: (i, 0)),
            pl.BlockSpec((TB, 1), lambda i: (i, 0)),
            pl.BlockSpec((NC, TB, 1), lambda i: (0, i, 0)),
            pl.BlockSpec((1, D), lambda i: (0, 0)),
            pl.BlockSpec((1, D), lambda i: (0, 0)),
            pl.BlockSpec((1, D), lambda i: (0, 0)),
            pl.BlockSpec((D, D), lambda i: (0, 0)),
            pl.BlockSpec((1, D), lambda i: (0, 0)),
            pl.BlockSpec((1, D), lambda i: (0, 0)),
            pl.BlockSpec((1, D), lambda i: (0, 0)),
        ],
        out_specs=pl.BlockSpec((1, D), lambda i: (0, 0)),
        out_shape=jax.ShapeDtypeStruct((1, D), jnp.float32),
        scratch_shapes=[pltpu.VMEM((1, D), jnp.float32)],
    )(acc, t2, dinv, gp, b2, g2, be2, W3, b3, g3, be3)


def kernel(x, edge_index, W1, b1, gamma1, beta1, W2, b2, gamma2, beta2,
           W3, b3, gamma3, beta3):
    row = lambda v: v.reshape(1, D)

    ei_flat = edge_index.reshape(2 * E)
    dinv_r, gp = _prep_sc(ei_flat)
    dinv = dinv_r.reshape(NPAD)[:N].reshape(N, 1)
    t1 = _tc1(dinv, x, W1)
    acc1 = _conv_sc(t1, ei_flat)
    t2 = _tc2(acc1, t1, dinv, W2, row(b1), row(gamma1), row(beta1))
    acc2 = _conv_sc(t2, ei_flat)
    out = _tc3(acc2, t2, dinv, gp.reshape(NC, NPAD)[:, :N].reshape(NC, N, 1),
               row(b2), row(gamma2), row(beta2), W3, row(b3), row(gamma3),
               row(beta3))
    return out


# speedup vs baseline: 1.0179x; 1.0179x over previous
import dataclasses
import functools

import jax
import jax.numpy as jnp
from jax import lax
from jax.experimental import pallas as pl
from jax.experimental.pallas import tpu as pltpu
from jax.experimental.pallas import tpu_sc as plsc

N = 10000
NPAD = 10240
E = 320000
D = 128
EPS = 1e-5
BNS = 1.0 / (1.0 + EPS) ** 0.5

NC, NS, L = 2, 16, 16
NW = NC * NS
EPW = E // NW
ST = 128
SUP = ST
SUPT = 78
NSUP = SUPT
NEXTRA = (E - NW * SUPT * SUP) // ST
HR = NPAD // 128
ICH = 2000
NICH = EPW // ICH

_MESH = plsc.VectorSubcoreMesh(core_axis_name="c", subcore_axis_name="s")
_CP = pltpu.CompilerParams()
if "needs_layout_passes" in pltpu.CompilerParams.__dataclass_fields__:
    _CP = dataclasses.replace(_CP, needs_layout_passes=False)

RPT_A = 624
RPT_B = N - 15 * RPT_A
TB = 400
NTB = N // TB


def _hist_body(weighted, ei_hbm, *rest):
    if weighted:
        (dinv_hbm, out_hbm, hist, shist, idxrows, ibuf_s, ibuf_d, dinv_v) = rest
    else:
        (out_hbm, hist, shist, idxrows, ibuf_s) = rest
        ibuf_d = ibuf_s
    c = lax.axis_index("c")
    s = lax.axis_index("s")
    w = c * NS + s
    base = w * EPW

    zf = jnp.zeros((L,), jnp.float32)
    lane_iota = lax.iota(jnp.int32, L)

    @pl.loop(0, HR)
    def _(i):
        @pl.loop(0, 128 // L)
        def _(k):
            hist[i, pl.ds(k * L, L)] = zf

    @pl.loop(0, HR // L)
    def _(k):
        idxrows[0, pl.ds(k * L, L)] = k * L + lane_iota

    @pl.when(s == 0)
    def _():
        pltpu.sync_copy(hist, shist)

    if weighted:
        pltpu.sync_copy(dinv_hbm, dinv_v)
    plsc.subcore_barrier()

    @pl.loop(0, NICH)
    def _(ci):
        if weighted:
            pltpu.sync_copy(ei_hbm.at[pl.ds(base + ci * ICH, ICH)], ibuf_s)
            pltpu.sync_copy(ei_hbm.at[pl.ds(E + base + ci * ICH, ICH)], ibuf_d)
        else:
            pltpu.sync_copy(ei_hbm.at[pl.ds(E + base + ci * ICH, ICH)], ibuf_s)

        @pl.loop(0, ICH // L)
        def _(i):
            key = ibuf_s[pl.ds(i * L, L)]
            if weighted:
                vals = plsc.load_gather(dinv_v, [ibuf_d[pl.ds(i * L, L)]])
            else:
                vals = zf + 1.0
            plsc.addupdate_scatter(
                hist,
                [lax.shift_right_logical(key, 7), lax.bitwise_and(key, 127)],
                vals,
            )

    pltpu.sync_copy(hist, shist.at[idxrows.at[0]], add=True)
    plsc.subcore_barrier()

    @pl.when(s == 0)
    def _():
        pltpu.sync_copy(shist, out_hbm.at[c])


def _deg_hist(ei):
    body = functools.partial(_hist_body, False)
    return pl.kernel(
        body,
        out_type=jax.ShapeDtypeStruct((NC, HR, 128), jnp.float32),
        mesh=_MESH,
        compiler_params=_CP,
        scratch_types=[
            pltpu.VMEM((HR, 128), jnp.float32),
            pltpu.VMEM_SHARED((HR, 128), jnp.float32),
            pltpu.VMEM((1, HR), jnp.int32),
            pltpu.VMEM((ICH,), jnp.int32),
        ],
    )(ei)


def _g_hist(ei, dinv_flat):
    body = functools.partial(_hist_body, True)
    return pl.kernel(
        body,
        out_type=jax.ShapeDtypeStruct((NC, HR, 128), jnp.float32),
        mesh=_MESH,
        compiler_params=_CP,
        scratch_types=[
            pltpu.VMEM((HR, 128), jnp.float32),
            pltpu.VMEM_SHARED((HR, 128), jnp.float32),
            pltpu.VMEM((1, HR), jnp.int32),
            pltpu.VMEM((ICH,), jnp.int32),
            pltpu.VMEM((ICH,), jnp.int32),
            pltpu.VMEM((N,), jnp.float32),
        ],
    )(ei, dinv_flat)


def _conv_body(t_hbm, ei_hbm, out_hbm, sidx, didx, rows, acc, isem, gsem, ssem):
    c = lax.axis_index("c")
    s = lax.axis_index("s")
    w = c * NS + s
    base = w * SUPT * SUP

    @pl.when(c == 0)
    def _():
        @pl.when(s < NS - 1)
        def _():
            pltpu.sync_copy(t_hbm.at[pl.ds(s * RPT_A, RPT_A)],
                            acc.at[pl.ds(s * RPT_A, RPT_A)])

        @pl.when(s == NS - 1)
        def _():
            pltpu.sync_copy(t_hbm.at[pl.ds(15 * RPT_A, RPT_B)],
                            acc.at[pl.ds(15 * RPT_A, RPT_B)])

    @pl.when(c == 1)
    def _():
        @pl.loop(0, SUP)
        def _(i):
            @pl.loop(0, D // L)
            def _(k):
                rows[0, i, pl.ds(k * L, L)] = jnp.zeros((L,), jnp.float32)

        @pl.loop(0, 4)
        def _(k):
            pltpu.sync_copy(rows.at[0],
                            acc.at[pl.ds(s * RPT_A + k * ST, ST)])

        @pl.when(s < NS - 1)
        def _():
            pltpu.sync_copy(rows.at[0, pl.ds(0, RPT_A - 4 * ST)],
                            acc.at[pl.ds(s * RPT_A + 4 * ST, RPT_A - 4 * ST)])

        @pl.when(s == NS - 1)
        def _():
            pltpu.sync_copy(rows.at[0],
                            acc.at[pl.ds(15 * RPT_A + 4 * ST, ST)])
    plsc.subcore_barrier()

    def start_idx(j, m):
        pltpu.async_copy(ei_hbm.at[pl.ds(base + j * SUP, SUP)],
                         sidx.at[m], isem.at[m])
        pltpu.async_copy(ei_hbm.at[pl.ds(E + base + j * SUP, ST)],
                         didx.at[m, 0], isem.at[m])

    def wait_idx(m):
        pltpu.make_async_copy(ei_hbm.at[pl.ds(0, SUP)], sidx.at[m],
                              isem.at[m]).wait()
        pltpu.make_async_copy(ei_hbm.at[pl.ds(0, ST)], didx.at[m, 0],
                              isem.at[m]).wait()

    def start_gather(m, r):
        pltpu.async_copy(t_hbm.at[sidx.at[m]], rows.at[r], gsem.at[r])

    def wait_gather(r):
        pltpu.make_async_copy(t_hbm.at[pl.ds(0, ST)], rows.at[r],
                              gsem.at[r]).wait()

    def start_scatter(m, r):
        pltpu.async_copy(rows.at[r], acc.at[didx.at[m, 0]], ssem.at[r],
                         add=True)

    def wait_scatter(m, r):
        pltpu.make_async_copy(rows.at[r], acc.at[didx.at[m, 0]],
                              ssem.at[r]).wait()

    start_idx(0, 0)
    start_idx(1, 1)

    @pl.loop(0, NSUP)
    def _(j):
        m = lax.rem(j, 4)
        r = lax.rem(j, 3)
        wait_idx(m)

        @pl.when(j >= 3)
        def _():
            wait_scatter(lax.rem(j + 1, 4), r)
        start_gather(m, r)

        @pl.when(j >= 2)
        def _():
            r2 = lax.rem(j + 1, 3)
            m2 = lax.rem(j + 2, 4)
            wait_gather(r2)
            start_scatter(m2, r2)

        @pl.when(j + 2 < NSUP)
        def _():
            start_idx(j + 2, lax.rem(j + 2, 4))

    wait_gather((NSUP - 2) % 3)
    start_scatter((NSUP - 2) % 4, (NSUP - 2) % 3)
    wait_gather((NSUP - 1) % 3)
    start_scatter((NSUP - 1) % 4, (NSUP - 1) % 3)
    wait_scatter((NSUP - 3) % 4, (NSUP - 3) % 3)
    wait_scatter((NSUP - 2) % 4, (NSUP - 2) % 3)
    wait_scatter((NSUP - 1) % 4, (NSUP - 1) % 3)

    @pl.when(w < NEXTRA)
    def _():
        xbase = NW * SUPT * SUP + w * ST
        pltpu.sync_copy(ei_hbm.at[pl.ds(xbase, ST)], sidx.at[0])
        pltpu.sync_copy(ei_hbm.at[pl.ds(E + xbase, ST)], didx.at[0, 0])
        pltpu.sync_copy(t_hbm.at[sidx.at[0]], rows.at[0])
        pltpu.sync_copy(rows.at[0], acc.at[didx.at[0, 0]], add=True)

    plsc.subcore_barrier()

    @pl.when(s < NS - 1)
    def _():
        pltpu.sync_copy(acc.at[pl.ds(s * RPT_A, RPT_A)],
                        out_hbm.at[c, pl.ds(s * RPT_A, RPT_A)])

    @pl.when(s == NS - 1)
    def _():
        pltpu.sync_copy(acc.at[pl.ds(15 * RPT_A, RPT_B)],
                        out_hbm.at[c, pl.ds(15 * RPT_A, RPT_B)])


def _conv_sc(t, ei):
    return pl.kernel(
        _conv_body,
        out_type=jax.ShapeDtypeStruct((NC, N, D), jnp.float32),
        mesh=_MESH,
        compiler_params=_CP,
        scratch_types=[
            pltpu.VMEM((4, SUP), jnp.int32),
            pltpu.VMEM((4, 1, ST), jnp.int32),
            pltpu.VMEM((3, SUP, D), jnp.float32),
            pltpu.VMEM_SHARED((N, D), jnp.float32),
            pltpu.SemaphoreType.DMA((4,)),
            pltpu.SemaphoreType.DMA((3,)),
            pltpu.SemaphoreType.DMA((3,)),
        ],
    )(t, ei)


def _tc1_body(hist_ref, x_ref, w1_ref, dinv_ref, t1_ref):
    deg = hist_ref[0] + hist_ref[1] + 1.0
    dinv = lax.rsqrt(deg)
    dinv_ref[...] = dinv
    t1_ref[...] = dinv * jnp.dot(x_ref[...], w1_ref[...],
                                 preferred_element_type=jnp.float32)


def _tc1(histp, xpad, W1):
    return pl.pallas_call(
        _tc1_body,
        grid=(NTB,),
        in_specs=[
            pl.BlockSpec((NC, TB, 1), lambda i: (0, i, 0)),
            pl.BlockSpec((TB, D), lambda i: (i, 0)),
            pl.BlockSpec((D, D), lambda i: (0, 0)),
        ],
        out_specs=[
            pl.BlockSpec((TB, 1), lambda i: (i, 0)),
            pl.BlockSpec((TB, D), lambda i: (i, 0)),
        ],
        out_shape=[
            jax.ShapeDtypeStruct((N, 1), jnp.float32),
            jax.ShapeDtypeStruct((N, D), jnp.float32),
        ],
    )(histp, xpad, W1)


def _tc2_body(acc_ref, dinv_ref, w2_ref, b1_ref, g1_ref, be1_ref,
              t2_ref):
    dinv = dinv_ref[...]
    u = acc_ref[0] + acc_ref[1]
    pre = dinv * u + b1_ref[...]
    h1 = jnp.maximum(pre * (g1_ref[...] * BNS) + be1_ref[...], 0.0)
    t2_ref[...] = dinv * jnp.dot(h1, w2_ref[...],
                                 preferred_element_type=jnp.float32)


def _tc2(acc, dinv, W2, b1, g1, be1):
    return pl.pallas_call(
        _tc2_body,
        grid=(NTB,),
        in_specs=[
            pl.BlockSpec((NC, TB, D), lambda i: (0, i, 0)),
            pl.BlockSpec((TB, 1), lambda i: (i, 0)),
            pl.BlockSpec((D, D), lambda i: (0, 0)),
            pl.BlockSpec((1, D), lambda i: (0, 0)),
            pl.BlockSpec((1, D), lambda i: (0, 0)),
            pl.BlockSpec((1, D), lambda i: (0, 0)),
        ],
        out_specs=pl.BlockSpec((TB, D), lambda i: (i, 0)),
        out_shape=jax.ShapeDtypeStruct((N, D), jnp.float32),
    )(acc, dinv, W2, b1, g1, be1)


def _tc3_body(acc_ref, dinv_ref, gp_ref, b2_ref, g2_ref, be2_ref,
              w3_ref, b3_ref, g3_ref, be3_ref, out_ref, ysum):
    i = pl.program_id(0)
    dinv = dinv_ref[...]
    u = acc_ref[0] + acc_ref[1]
    pre = dinv * u + b2_ref[...]
    h2 = jnp.maximum(pre * (g2_ref[...] * BNS) + be2_ref[...], 0.0)
    g = gp_ref[0] + gp_ref[1]
    w = dinv * (g + dinv) * (1.0 / N)
    part = jnp.sum(w * h2, axis=0, keepdims=True)

    @pl.when(i == 0)
    def _():
        ysum[...] = jnp.zeros_like(ysum)

    ysum[...] += part

    @pl.when(i == NTB - 1)
    def _():
        y = jnp.dot(ysum[...], w3_ref[...], preferred_element_type=jnp.float32)
        out_ref[...] = (y + b3_ref[...]) * (g3_ref[...] * BNS) + be3_ref[...]


def _tc3(acc, dinv, gp, b2, g2, be2, W3, b3, g3, be3):
    return pl.pallas_call(
        _tc3_body,
        grid=(NTB,),
        in_specs=[
            pl.BlockSpec((NC, TB, D), lambda i: (0, i, 0)),
            pl.BlockSpec((TB, 1), lambda i: (i, 0)),
            pl.BlockSpec((NC, TB, 1), lambda i: (0, i, 0)),
            pl.BlockSpec((1, D), lambda i: (0, 0)),
            pl.BlockSpec((1, D), lambda i: (0, 0)),
            pl.BlockSpec((1, D), lambda i: (0, 0)),
            pl.BlockSpec((D, D), lambda i: (0, 0)),
            pl.BlockSpec((1, D), lambda i: (0, 0)),
            pl.BlockSpec((1, D), lambda i: (0, 0)),
            pl.BlockSpec((1, D), lambda i: (0, 0)),
        ],
        out_specs=pl.BlockSpec((1, D), lambda i: (0, 0)),
        out_shape=jax.ShapeDtypeStruct((1, D), jnp.float32),
        scratch_shapes=[pltpu.VMEM((1, D), jnp.float32)],
    )(acc, dinv, gp, b2, g2, be2, W3, b3, g3, be3)


def kernel(x, edge_index, W1, b1, gamma1, beta1, W2, b2, gamma2, beta2,
           W3, b3, gamma3, beta3):
    row = lambda v: v.reshape(1, D)

    ei_flat = edge_index.reshape(2 * E)
    histp = _deg_hist(ei_flat).reshape(NC, NPAD)[:, :N].reshape(NC, N, 1)
    dinv, t1 = _tc1(histp, x, W1)
    acc1 = _conv_sc(t1, ei_flat)
    gp = _g_hist(ei_flat, dinv.reshape(N))
    t2 = _tc2(acc1, dinv, W2, row(b1), row(gamma1), row(beta1))
    acc2 = _conv_sc(t2, ei_flat)
    out = _tc3(acc2, dinv, gp.reshape(NC, NPAD)[:, :N].reshape(NC, N, 1),
               row(b2), row(gamma2), row(beta2), W3, row(b3), row(gamma3),
               row(beta3))
    return out
